# baseline (speedup 1.0000x reference)
import functools

import jax
import jax.numpy as jnp
from jax import lax
from jax.experimental import pallas as pl
from jax.experimental.pallas import tpu as pltpu
from jax.experimental.pallas import tpu_sc as plsc

N = 10000
E = 320000
D = 128
G = 64

NC = 2
NS = 16
NW = NC * NS
CH = 64
NCHT = 160
HCH = 32
NST = 10
ACH = NST * HCH
E_PAD = NW * NCHT * CH
N_PAD = 10240
RPT = N_PAD // NS
BLK = 256
NBLK = N_PAD // BLK

@functools.cache
def _sc_mesh():
  return plsc.VectorSubcoreMesh(
      core_axis_name="c", subcore_axis_name="s", num_cores=NC, num_subcores=NS)


def _fill_rows(ref, nrows, ncols, value):
  nv = ncols // 16
  def body(i, _):
    r = i // nv
    k = i % nv
    ref[r, pl.ds(k * 16, 16)] = jnp.full((16,), value, jnp.float32)
    return 0
  lax.fori_loop(0, nrows * nv, body, 0)


def _deg_body(dst_hbm, out_hbm, idx_v, ones_v, acc_sh):
  c = lax.axis_index("c")
  s = lax.axis_index("s")
  wid = s * NC + c
  _fill_rows(ones_v, CH, D, 0.0)
  for k in range(RPT // CH):
    pltpu.sync_copy(ones_v, acc_sh.at[pl.ds(s * RPT + k * CH, CH)])
  _fill_rows(ones_v, CH, D, 1.0)
  pltpu.sync_copy(dst_hbm.at[pl.ds(wid * NCHT, NCHT)], idx_v)
  plsc.subcore_barrier()
  def body(j, _):
    pltpu.sync_copy(ones_v, acc_sh.at[idx_v.at[j]], add=True)
    return 0
  lax.fori_loop(0, NCHT, body, 0)
  plsc.subcore_barrier()
  pltpu.sync_copy(acc_sh.at[pl.ds(s * RPT, RPT)],
                  out_hbm.at[c].at[pl.ds(s * RPT, RPT)])


@functools.cache
def _deg_kernel():
  return pl.kernel(
      _deg_body,
      out_type=jax.ShapeDtypeStruct((NC, N_PAD, D), jnp.float32),
      mesh=_sc_mesh(),
      scratch_types=[
          pltpu.VMEM((NCHT, CH), jnp.int32),
          pltpu.VMEM((CH, D), jnp.float32),
          pltpu.VMEM_SHARED((N_PAD, D), jnp.float32),
      ],
  )


NBUF = 4


def _agg_body(p_hbm, src_hbm, dst_hbm, out_hbm, sidx_v, didx_v, rows0_v,
              rows1_v, rows2_v, rows3_v, acc_sh, sem0, sem1, sem2, sem3):
  c = lax.axis_index("c")
  s = lax.axis_index("s")

  @pl.when(c == 0)
  def _():
    _fill_rows(rows0_v, CH, D, 0.0)
    for k in range(RPT // CH):
      pltpu.sync_copy(rows0_v, acc_sh.at[pl.ds(s * RPT + k * CH, CH)])
  plsc.subcore_barrier()

  bufs = ((rows0_v, sem0), (rows1_v, sem1), (rows2_v, sem2), (rows3_v, sem3))
  tile_base = s * ACH
  for h in range(NST):
    @pl.when(c == 0)
    def _():
      pltpu.sync_copy(src_hbm.at[pl.ds(tile_base + h * HCH, HCH)], sidx_v)
      pltpu.sync_copy(dst_hbm.at[pl.ds(tile_base + h * HCH, HCH)], didx_v)
      for b, (buf, sem) in enumerate(bufs):
        pltpu.async_copy(p_hbm.at[sidx_v.at[b]], buf, sem)

      def grp(g, _):
        for b, (buf, sem) in enumerate(bufs):
          j = g * NBUF + b
          pltpu.make_async_copy(p_hbm.at[sidx_v.at[j]], buf, sem).wait()
          pltpu.sync_copy(buf, acc_sh.at[didx_v.at[j]], add=True)

          @pl.when(j + NBUF < HCH)
          def _():
            pltpu.async_copy(p_hbm.at[sidx_v.at[j + NBUF]], buf, sem)
        return 0

      lax.fori_loop(0, HCH // NBUF, grp, 0)
  plsc.subcore_barrier()

  @pl.when(c == 0)
  def _():
    pltpu.sync_copy(acc_sh.at[pl.ds(s * RPT, RPT)],
                    out_hbm.at[pl.ds(s * RPT, RPT)])


@functools.cache
def _agg_kernel():
  return pl.kernel(
      _agg_body,
      out_type=jax.ShapeDtypeStruct((N_PAD, D), jnp.float32),
      mesh=_sc_mesh(),
      scratch_types=[
          pltpu.VMEM((HCH, CH), jnp.int32),
          pltpu.VMEM((HCH, CH), jnp.int32),
          pltpu.VMEM((CH, D), jnp.float32),
          pltpu.VMEM((CH, D), jnp.float32),
          pltpu.VMEM((CH, D), jnp.float32),
          pltpu.VMEM((CH, D), jnp.float32),
          pltpu.VMEM_SHARED((N_PAD, D), jnp.float32),
          pltpu.SemaphoreType.DMA,
          pltpu.SemaphoreType.DMA,
          pltpu.SemaphoreType.DMA,
          pltpu.SemaphoreType.DMA,
      ],
  )


def _dinv_body(d0_ref, d1_ref, o_ref):
  deg = d0_ref[:, 0:1] + d1_ref[:, 0:1] + 1.0
  o_ref[...] = jnp.broadcast_to(lax.rsqrt(deg), (BLK, D))


def _dinv_call(d0, d1):
  return pl.pallas_call(
      _dinv_body,
      grid=(NBLK,),
      in_specs=[pl.BlockSpec((BLK, D), lambda j: (j, 0)),
                pl.BlockSpec((BLK, D), lambda j: (j, 0))],
      out_specs=pl.BlockSpec((BLK, D), lambda j: (j, 0)),
      out_shape=jax.ShapeDtypeStruct((N_PAD, D), jnp.float32),
  )(d0, d1)


def _m1_body(x_ref, w_ref, dinv_ref, o_ref):
  o_ref[...] = jnp.dot(x_ref[...], w_ref[...],
                       preferred_element_type=jnp.float32) * dinv_ref[...]


def _m1_call(x, w, dinv):
  return pl.pallas_call(
      _m1_body,
      grid=(NBLK,),
      in_specs=[pl.BlockSpec((BLK, D), lambda j: (j, 0)),
                pl.BlockSpec((D, D), lambda j: (0, 0)),
                pl.BlockSpec((BLK, D), lambda j: (j, 0))],
      out_specs=pl.BlockSpec((BLK, D), lambda j: (j, 0)),
      out_shape=jax.ShapeDtypeStruct((N_PAD, D), jnp.float32),
  )(x, w, dinv)


def _m2_body(a_ref, p_ref, dinv_ref, b_ref, w_ref, o_ref):
  h = jnp.maximum(
      (a_ref[...] + p_ref[...]) * dinv_ref[...] + b_ref[...], 0.0)
  o_ref[...] = jnp.dot(h, w_ref[...],
                       preferred_element_type=jnp.float32) * dinv_ref[...]


def _m2_call(a, p, dinv, b, w):
  return pl.pallas_call(
      _m2_body,
      grid=(NBLK,),
      in_specs=[pl.BlockSpec((BLK, D), lambda j: (j, 0)),
                pl.BlockSpec((BLK, D), lambda j: (j, 0)),
                pl.BlockSpec((BLK, D), lambda j: (j, 0)),
                pl.BlockSpec((1, D), lambda j: (0, 0)),
                pl.BlockSpec((D, D), lambda j: (0, 0))],
      out_specs=pl.BlockSpec((BLK, D), lambda j: (j, 0)),
      out_shape=jax.ShapeDtypeStruct((N_PAD, D), jnp.float32),
  )(a, p, dinv, b, w)


def _final_body(a_ref, p_ref, dinv_ref, b2_ref, batch_ref, lw0_ref,
                lb0_ref, lw1_ref, lb1_ref, emb_ref, prob_ref, sums, counts):
  j = pl.program_id(0)

  @pl.when(j == 0)
  def _():
    sums[...] = jnp.zeros_like(sums)
    counts[...] = jnp.zeros_like(counts)

  h3 = jnp.maximum(
      (a_ref[...] + p_ref[...]) * dinv_ref[...] + b2_ref[...], 0.0)
  ids = batch_ref[0, 0, :]
  oh_t = (lax.broadcasted_iota(jnp.int32, (G, BLK), 0)
          == ids[None, :]).astype(jnp.float32)
  sums[...] += jnp.dot(oh_t, h3, preferred_element_type=jnp.float32)
  counts[...] += jnp.broadcast_to(jnp.sum(oh_t, axis=1)[:, None], (G, D))

  @pl.when(j == NBLK - 1)
  def _():
    emb = sums[...] / jnp.maximum(counts[...], 1.0)
    emb_ref[...] = emb
    z = jnp.maximum(
        jnp.dot(emb, lw0_ref[...], preferred_element_type=jnp.float32)
        + lb0_ref[...], 0.0)
    logits = jnp.dot(z, lw1_ref[...],
                     preferred_element_type=jnp.float32) + lb1_ref[...]
    prob_ref[...] = jax.nn.sigmoid(logits)


def _final_call(a, p, dinv, b2, batchp, lw0, lb0, lw1p, lb1p):
  return pl.pallas_call(
      _final_body,
      grid=(NBLK,),
      in_specs=[pl.BlockSpec((BLK, D), lambda j: (j, 0)),
                pl.BlockSpec((BLK, D), lambda j: (j, 0)),
                pl.BlockSpec((BLK, D), lambda j: (j, 0)),
                pl.BlockSpec((1, D), lambda j: (0, 0)),
                pl.BlockSpec((1, 1, BLK), lambda j: (j, 0, 0)),
                pl.BlockSpec((D, D), lambda j: (0, 0)),
                pl.BlockSpec((1, D), lambda j: (0, 0)),
                pl.BlockSpec((D, D), lambda j: (0, 0)),
                pl.BlockSpec((1, D), lambda j: (0, 0))],
      out_specs=[pl.BlockSpec((G, D), lambda j: (0, 0)),
                 pl.BlockSpec((G, D), lambda j: (0, 0))],
      out_shape=[jax.ShapeDtypeStruct((G, D), jnp.float32),
                 jax.ShapeDtypeStruct((G, D), jnp.float32)],
      scratch_shapes=[pltpu.VMEM((G, D), jnp.float32),
                      pltpu.VMEM((G, D), jnp.float32)],
  )(a, p, dinv, b2, batchp, lw0, lb0, lw1p, lb1p)


def kernel(x, edge_index, batch, W0, b0, W1, b1, W2, b2, lW0, lb0, lW1, lb1):
  pad_e = E_PAD - E
  srcp = jnp.concatenate(
      [edge_index[0], jnp.full((pad_e,), N, jnp.int32)]).reshape(-1, CH)
  dstp = jnp.concatenate(
      [edge_index[1], jnp.full((pad_e,), N, jnp.int32)]).reshape(-1, CH)
  xp = jnp.pad(x, ((0, N_PAD - N), (0, 0)))
  batchp = jnp.pad(batch, (0, N_PAD - N),
                   constant_values=G).reshape(NBLK, 1, BLK)

  degp = _deg_kernel()(dstp)
  dinv = _dinv_call(degp[0], degp[1])

  p0 = _m1_call(xp, W0, dinv)
  acc = _agg_kernel()(p0, srcp, dstp)
  p1 = _m2_call(acc, p0, dinv, b0.reshape(1, D), W1)
  acc = _agg_kernel()(p1, srcp, dstp)
  p2 = _m2_call(acc, p1, dinv, b1.reshape(1, D), W2)
  acc = _agg_kernel()(p2, srcp, dstp)

  lW1p = jnp.pad(lW1, ((0, 0), (0, D - 1)))
  lb1p = jnp.pad(lb1.reshape(1, 1), ((0, 0), (0, D - 1)))
  emb, probs = _final_call(acc, p2, dinv, b2.reshape(1, D),
                           batchp, lW0, lb0.reshape(1, D), lW1p, lb1p)
  return probs[:, 0], emb


# speedup vs baseline: 3.1242x; 3.1242x over previous
import functools

import jax
import jax.numpy as jnp
from jax import lax
from jax.experimental import pallas as pl
from jax.experimental.pallas import tpu as pltpu
from jax.experimental.pallas import tpu_sc as plsc

N = 10000
E = 320000
D = 128
G = 64

NC = 2
NS = 16
NW = NC * NS
CH = 64
NCHT = 160
HCH = 32
NST = NCHT // HCH
E_PAD = NW * NCHT * CH
N_PAD = 10240
RPT = N_PAD // NS
BLK = 256
NBLK = N_PAD // BLK

@functools.cache
def _sc_mesh():
  return plsc.VectorSubcoreMesh(
      core_axis_name="c", subcore_axis_name="s", num_cores=NC, num_subcores=NS)


def _fill_rows(ref, nrows, ncols, value):
  nv = ncols // 16
  def body(i, _):
    r = i // nv
    k = i % nv
    ref[r, pl.ds(k * 16, 16)] = jnp.full((16,), value, jnp.float32)
    return 0
  lax.fori_loop(0, nrows * nv, body, 0)


def _deg_body(dst_hbm, out_hbm, idx_v, ones_v, acc_sh):
  c = lax.axis_index("c")
  s = lax.axis_index("s")
  wid = s * NC + c
  _fill_rows(ones_v, CH, D, 0.0)
  for k in range(RPT // CH):
    pltpu.sync_copy(ones_v, acc_sh.at[pl.ds(s * RPT + k * CH, CH)])
  _fill_rows(ones_v, CH, D, 1.0)
  pltpu.sync_copy(dst_hbm.at[pl.ds(wid * NCHT, NCHT)], idx_v)
  plsc.subcore_barrier()
  def body(j, _):
    pltpu.sync_copy(ones_v, acc_sh.at[idx_v.at[j]], add=True)
    return 0
  lax.fori_loop(0, NCHT, body, 0)
  plsc.subcore_barrier()
  pltpu.sync_copy(acc_sh.at[pl.ds(s * RPT, RPT)],
                  out_hbm.at[c].at[pl.ds(s * RPT, RPT)])


@functools.cache
def _deg_kernel():
  return pl.kernel(
      _deg_body,
      out_type=jax.ShapeDtypeStruct((NC, N_PAD, D), jnp.float32),
      mesh=_sc_mesh(),
      scratch_types=[
          pltpu.VMEM((NCHT, CH), jnp.int32),
          pltpu.VMEM((CH, D), jnp.float32),
          pltpu.VMEM_SHARED((N_PAD, D), jnp.float32),
      ],
  )


NBUF = 4


def _agg_body(p_hbm, src_hbm, dst_hbm, out_hbm, sidx_v, didx_v, rows0_v,
              rows1_v, rows2_v, rows3_v, acc_sh, sem0, sem1, sem2, sem3):
  c = lax.axis_index("c")
  s = lax.axis_index("s")
  wid = s * NC + c
  _fill_rows(rows0_v, CH, D, 0.0)
  for k in range(RPT // CH):
    pltpu.sync_copy(rows0_v, acc_sh.at[pl.ds(s * RPT + k * CH, CH)])
  plsc.subcore_barrier()

  bufs = ((rows0_v, sem0), (rows1_v, sem1), (rows2_v, sem2), (rows3_v, sem3))
  tile_base = wid * NCHT
  for h in range(NST):
    pltpu.sync_copy(src_hbm.at[pl.ds(tile_base + h * HCH, HCH)], sidx_v)
    pltpu.sync_copy(dst_hbm.at[pl.ds(tile_base + h * HCH, HCH)], didx_v)
    for b, (buf, sem) in enumerate(bufs):
      pltpu.async_copy(p_hbm.at[sidx_v.at[b]], buf, sem)

    def grp(g, _):
      for b, (buf, sem) in enumerate(bufs):
        j = g * NBUF + b
        pltpu.make_async_copy(p_hbm.at[sidx_v.at[j]], buf, sem).wait()
        pltpu.sync_copy(buf, acc_sh.at[didx_v.at[j]], add=True)

        @pl.when(j + NBUF < HCH)
        def _():
          pltpu.async_copy(p_hbm.at[sidx_v.at[j + NBUF]], buf, sem)
      return 0

    lax.fori_loop(0, HCH // NBUF, grp, 0)
  plsc.subcore_barrier()
  pltpu.sync_copy(acc_sh.at[pl.ds(s * RPT, RPT)],
                  out_hbm.at[c].at[pl.ds(s * RPT, RPT)])


@functools.cache
def _agg_kernel():
  return pl.kernel(
      _agg_body,
      out_type=jax.ShapeDtypeStruct((NC, N_PAD, D), jnp.float32),
      mesh=_sc_mesh(),
      scratch_types=[
          pltpu.VMEM((HCH, CH), jnp.int32),
          pltpu.VMEM((HCH, CH), jnp.int32),
          pltpu.VMEM((CH, D), jnp.float32),
          pltpu.VMEM((CH, D), jnp.float32),
          pltpu.VMEM((CH, D), jnp.float32),
          pltpu.VMEM((CH, D), jnp.float32),
          pltpu.VMEM_SHARED((N_PAD, D), jnp.float32),
          pltpu.SemaphoreType.DMA,
          pltpu.SemaphoreType.DMA,
          pltpu.SemaphoreType.DMA,
          pltpu.SemaphoreType.DMA,
      ],
  )


def _dinv_body(d0_ref, d1_ref, o_ref):
  deg = d0_ref[:, 0:1] + d1_ref[:, 0:1] + 1.0
  o_ref[...] = jnp.broadcast_to(lax.rsqrt(deg), (BLK, D))


def _dinv_call(d0, d1):
  return pl.pallas_call(
      _dinv_body,
      grid=(NBLK,),
      in_specs=[pl.BlockSpec((BLK, D), lambda j: (j, 0)),
                pl.BlockSpec((BLK, D), lambda j: (j, 0))],
      out_specs=pl.BlockSpec((BLK, D), lambda j: (j, 0)),
      out_shape=jax.ShapeDtypeStruct((N_PAD, D), jnp.float32),
  )(d0, d1)


def _m1_body(x_ref, w_ref, dinv_ref, o_ref):
  o_ref[...] = jnp.dot(x_ref[...], w_ref[...],
                       preferred_element_type=jnp.float32) * dinv_ref[...]


def _m1_call(x, w, dinv):
  return pl.pallas_call(
      _m1_body,
      grid=(NBLK,),
      in_specs=[pl.BlockSpec((BLK, D), lambda j: (j, 0)),
                pl.BlockSpec((D, D), lambda j: (0, 0)),
                pl.BlockSpec((BLK, D), lambda j: (j, 0))],
      out_specs=pl.BlockSpec((BLK, D), lambda j: (j, 0)),
      out_shape=jax.ShapeDtypeStruct((N_PAD, D), jnp.float32),
  )(x, w, dinv)


def _m2_body(a0_ref, a1_ref, p_ref, dinv_ref, b_ref, w_ref, o_ref):
  h = jnp.maximum(
      (a0_ref[...] + a1_ref[...] + p_ref[...]) * dinv_ref[...] + b_ref[...],
      0.0)
  o_ref[...] = jnp.dot(h, w_ref[...],
                       preferred_element_type=jnp.float32) * dinv_ref[...]


def _m2_call(a, p, dinv, b, w):
  return pl.pallas_call(
      _m2_body,
      grid=(NBLK,),
      in_specs=[pl.BlockSpec((BLK, D), lambda j: (j, 0)),
                pl.BlockSpec((BLK, D), lambda j: (j, 0)),
                pl.BlockSpec((BLK, D), lambda j: (j, 0)),
                pl.BlockSpec((BLK, D), lambda j: (j, 0)),
                pl.BlockSpec((1, D), lambda j: (0, 0)),
                pl.BlockSpec((D, D), lambda j: (0, 0))],
      out_specs=pl.BlockSpec((BLK, D), lambda j: (j, 0)),
      out_shape=jax.ShapeDtypeStruct((N_PAD, D), jnp.float32),
  )(a[0], a[1], p, dinv, b, w)


def _final_body(a0_ref, a1_ref, p_ref, dinv_ref, b2_ref, batch_ref, lw0_ref,
                lb0_ref, lw1_ref, lb1_ref, emb_ref, prob_ref, sums, counts):
  j = pl.program_id(0)

  @pl.when(j == 0)
  def _():
    sums[...] = jnp.zeros_like(sums)
    counts[...] = jnp.zeros_like(counts)

  h3 = jnp.maximum(
      (a0_ref[...] + a1_ref[...] + p_ref[...]) * dinv_ref[...] + b2_ref[...],
      0.0)
  ids = batch_ref[0, 0, :]
  oh_t = (lax.broadcasted_iota(jnp.int32, (G, BLK), 0)
          == ids[None, :]).astype(jnp.float32)
  sums[...] += jnp.dot(oh_t, h3, preferred_element_type=jnp.float32)
  counts[...] += jnp.broadcast_to(jnp.sum(oh_t, axis=1)[:, None], (G, D))

  @pl.when(j == NBLK - 1)
  def _():
    emb = sums[...] / jnp.maximum(counts[...], 1.0)
    emb_ref[...] = emb
    z = jnp.maximum(
        jnp.dot(emb, lw0_ref[...], preferred_element_type=jnp.float32)
        + lb0_ref[...], 0.0)
    logits = jnp.dot(z, lw1_ref[...],
                     preferred_element_type=jnp.float32) + lb1_ref[...]
    prob_ref[...] = jax.nn.sigmoid(logits)


def _final_call(a, p, dinv, b2, batchp, lw0, lb0, lw1p, lb1p):
  return pl.pallas_call(
      _final_body,
      grid=(NBLK,),
      in_specs=[pl.BlockSpec((BLK, D), lambda j: (j, 0)),
                pl.BlockSpec((BLK, D), lambda j: (j, 0)),
                pl.BlockSpec((BLK, D), lambda j: (j, 0)),
                pl.BlockSpec((BLK, D), lambda j: (j, 0)),
                pl.BlockSpec((1, D), lambda j: (0, 0)),
                pl.BlockSpec((1, 1, BLK), lambda j: (j, 0, 0)),
                pl.BlockSpec((D, D), lambda j: (0, 0)),
                pl.BlockSpec((1, D), lambda j: (0, 0)),
                pl.BlockSpec((D, D), lambda j: (0, 0)),
                pl.BlockSpec((1, D), lambda j: (0, 0))],
      out_specs=[pl.BlockSpec((G, D), lambda j: (0, 0)),
                 pl.BlockSpec((G, D), lambda j: (0, 0))],
      out_shape=[jax.ShapeDtypeStruct((G, D), jnp.float32),
                 jax.ShapeDtypeStruct((G, D), jnp.float32)],
      scratch_shapes=[pltpu.VMEM((G, D), jnp.float32),
                      pltpu.VMEM((G, D), jnp.float32)],
  )(a[0], a[1], p, dinv, b2, batchp, lw0, lb0, lw1p, lb1p)


def kernel(x, edge_index, batch, W0, b0, W1, b1, W2, b2, lW0, lb0, lW1, lb1):
  pad_e = E_PAD - E
  pad_ids = N + (jnp.arange(pad_e, dtype=jnp.int32) % (N_PAD - N))
  srcp = jnp.concatenate([edge_index[0], pad_ids]).reshape(-1, CH)
  dstp = jnp.concatenate([edge_index[1], pad_ids]).reshape(-1, CH)
  xp = jnp.pad(x, ((0, N_PAD - N), (0, 0)))
  batchp = jnp.pad(batch, (0, N_PAD - N),
                   constant_values=G).reshape(NBLK, 1, BLK)

  degp = _deg_kernel()(dstp)
  dinv = _dinv_call(degp[0], degp[1])

  p0 = _m1_call(xp, W0, dinv)
  acc = _agg_kernel()(p0, srcp, dstp)
  p1 = _m2_call(acc, p0, dinv, b0.reshape(1, D), W1)
  acc = _agg_kernel()(p1, srcp, dstp)
  p2 = _m2_call(acc, p1, dinv, b1.reshape(1, D), W2)
  acc = _agg_kernel()(p2, srcp, dstp)

  lW1p = jnp.pad(lW1, ((0, 0), (0, D - 1)))
  lb1p = jnp.pad(lb1.reshape(1, 1), ((0, 0), (0, D - 1)))
  emb, probs = _final_call(acc, p2, dinv, b2.reshape(1, D),
                           batchp, lW0, lb0.reshape(1, D), lW1p, lb1p)
  return probs[:, 0], emb
